# Initial kernel scaffold; baseline (speedup 1.0000x reference)
import jax
import jax.numpy as jnp
from jax.experimental import pallas as pl


def kernel(x, edge_index, W1, b1, W2, b2, W3, b3, Wfc, bfc):
    raise NotImplementedError("write your pallas kernel here")


# speedup vs baseline: 134.9899x; 134.9899x over previous
import functools

import jax
import jax.numpy as jnp
from jax import lax
from jax.experimental import pallas as pl
from jax.experimental.pallas import tpu as pltpu
from jax.experimental.pallas import tpu_sc as plsc

N = 10000
E = 320000
NW = 32
EW = E // NW
CH = 128
NCH = 80
EWP = CH * NCH
EP = EWP * NW
ACC_ROWS = 10240
PAD_BIN = 10016
HB = 160
RB = 400
F32 = jnp.float32

def _z():
    return jnp.int32(0)


_mesh = plsc.VectorSubcoreMesh(core_axis_name="c", subcore_axis_name="s")


NBINS = HB * 64


@functools.partial(
    pl.kernel,
    mesh=_mesh,
    out_type=[
        jax.ShapeDtypeStruct((NW, NBINS), F32),
        jax.ShapeDtypeStruct((NW, NBINS), F32),
    ],
    scratch_types=[
        pltpu.VMEM((EW,), jnp.int32),
        pltpu.VMEM((EW,), jnp.int32),
        pltpu.VMEM((NBINS,), F32),
        pltpu.VMEM((NBINS,), F32),
    ],
    compiler_params=pltpu.CompilerParams(needs_layout_passes=False),
)
def _deg_kernel(src_hbm, dst_hbm, zeros_hbm, odeg_hbm, ideg_hbm,
                src_v, dst_v, ho_v, hi_v):
    c = lax.axis_index("c")
    s = lax.axis_index("s")
    wid = s * 2 + c
    pltpu.sync_copy(zeros_hbm, ho_v)
    pltpu.sync_copy(zeros_hbm, hi_v)
    pltpu.sync_copy(src_hbm.at[pl.ds(wid * EW, EW)], src_v)
    pltpu.sync_copy(dst_hbm.at[pl.ds(wid * EW, EW)], dst_v)
    ones = jnp.ones((16,), F32)

    def body(i, carry):
        sv = src_v[pl.ds(i * 16, 16)]
        dv = dst_v[pl.ds(i * 16, 16)]
        plsc.addupdate_scatter(ho_v, [sv], ones)
        plsc.addupdate_scatter(hi_v, [dv], ones)
        return carry

    lax.fori_loop(jnp.int32(0), jnp.int32(EW // 16), body, jnp.int32(0))
    pltpu.sync_copy(ho_v, odeg_hbm.at[wid])
    pltpu.sync_copy(hi_v, ideg_hbm.at[wid])


@functools.partial(
    pl.kernel,
    mesh=_mesh,
    out_type=jax.ShapeDtypeStruct((2, ACC_ROWS, 64), F32),
    scratch_types=[
        pltpu.VMEM((NCH, CH), jnp.int32),
        pltpu.VMEM((NCH, CH), jnp.int32),
        pltpu.VMEM((CH, 64), F32),
        pltpu.VMEM_SHARED((ACC_ROWS, 64), F32),
        pltpu.SemaphoreType.DMA,
    ],
    compiler_params=pltpu.CompilerParams(
        needs_layout_passes=False, use_tc_tiling_on_sc=False),
)
def _agg_kernel(hw_hbm, srcp_hbm, dstp_hbm, zacc_hbm, parts_hbm,
                src_v, dst_v, rows_v, acc, sem):
    c = lax.axis_index("c")
    s = lax.axis_index("s")
    wid = s * 2 + c
    rows_per_tile = ACC_ROWS // 16
    sl = pl.ds(s * rows_per_tile, rows_per_tile)
    pltpu.sync_copy(zacc_hbm.at[sl], acc.at[sl])
    pltpu.sync_copy(srcp_hbm.at[wid], src_v)
    pltpu.sync_copy(dstp_hbm.at[wid], dst_v)
    plsc.subcore_barrier()

    def body(j, carry):
        pltpu.async_copy(hw_hbm.at[src_v.at[j]], rows_v, sem).wait()
        pltpu.sync_copy(rows_v, acc.at[dst_v.at[j]], add=True)
        return carry

    lax.fori_loop(jnp.int32(0), jnp.int32(NCH), body, jnp.int32(0))
    plsc.subcore_barrier()
    pltpu.sync_copy(acc.at[sl], parts_hbm.at[c, sl])


def _norms_body(od_ref, id_ref, onorm_ref, inorm_ref):
    od = jnp.sum(od_ref[...], axis=0, keepdims=True)
    idg = jnp.sum(id_ref[...], axis=0, keepdims=True)
    onorm_ref[...] = jnp.where(od > 0, lax.rsqrt(jnp.maximum(od, 1.0)), 0.0)
    inorm_ref[...] = jnp.where(idg > 0, lax.rsqrt(jnp.maximum(idg, 1.0)), 0.0)


def _norms(odeg_p, ideg_p):
    return pl.pallas_call(
        _norms_body,
        out_shape=[
            jax.ShapeDtypeStruct((1, NBINS), F32),
            jax.ShapeDtypeStruct((1, NBINS), F32),
        ],
    )(odeg_p, ideg_p)


def _proj1_body(x_ref, w_ref, on_ref, o_ref):
    o_ref[...] = lax.dot(
        x_ref[...], w_ref[...], preferred_element_type=F32) * on_ref[...]


def _proj1(x, W1, onorm):
    grid = N // RB
    return pl.pallas_call(
        _proj1_body,
        grid=(grid,),
        in_specs=[
            pl.BlockSpec((RB, 128), lambda i: (i, _z())),
            pl.BlockSpec((128, 64), lambda i: (_z(), _z())),
            pl.BlockSpec((RB, 1), lambda i: (i, _z())),
        ],
        out_specs=pl.BlockSpec((RB, 64), lambda i: (i, _z())),
        out_shape=jax.ShapeDtypeStruct((N, 64), F32),
    )(x, W1, onorm)


def _mid_body(p_ref, in_ref, b_ref, w_ref, on_ref, o_ref):
    h = jnp.tanh((p_ref[0] + p_ref[1]) * in_ref[...] + b_ref[...])
    o_ref[...] = lax.dot(
        h, w_ref[...], preferred_element_type=F32) * on_ref[...]


def _mid(parts, inorm, b, W, onorm):
    grid = N // RB
    return pl.pallas_call(
        _mid_body,
        grid=(grid,),
        in_specs=[
            pl.BlockSpec((2, RB, 64), lambda i: (_z(), i, _z())),
            pl.BlockSpec((RB, 1), lambda i: (i, _z())),
            pl.BlockSpec((1, 64), lambda i: (_z(), _z())),
            pl.BlockSpec((64, 64), lambda i: (_z(), _z())),
            pl.BlockSpec((RB, 1), lambda i: (i, _z())),
        ],
        out_specs=pl.BlockSpec((RB, 64), lambda i: (i, _z())),
        out_shape=jax.ShapeDtypeStruct((N, 64), F32),
    )(parts, inorm, b, W, onorm)


def _head_body(p_ref, in_ref, b_ref, w_ref, bf_ref, o_ref):
    h = jnp.tanh((p_ref[0] + p_ref[1]) * in_ref[...] + b_ref[...])
    o_ref[...] = lax.dot(
        h, w_ref[...], preferred_element_type=F32) + bf_ref[...]


def _head(parts, inorm, b, Wfc, bfc):
    grid = N // RB
    return pl.pallas_call(
        _head_body,
        grid=(grid,),
        in_specs=[
            pl.BlockSpec((2, RB, 64), lambda i: (_z(), i, _z())),
            pl.BlockSpec((RB, 1), lambda i: (i, _z())),
            pl.BlockSpec((1, 64), lambda i: (_z(), _z())),
            pl.BlockSpec((64, 10), lambda i: (_z(), _z())),
            pl.BlockSpec((1, 10), lambda i: (_z(), _z())),
        ],
        out_specs=pl.BlockSpec((RB, 10), lambda i: (i, _z())),
        out_shape=jax.ShapeDtypeStruct((N, 10), F32),
    )(parts, inorm, b, Wfc, bfc)


def kernel(x, edge_index, W1, b1, W2, b2, W3, b3, Wfc, bfc):
    out_dtype = jnp.result_type(x.dtype, W1.dtype)
    x, W1, b1, W2, b2, W3, b3, Wfc, bfc = (
        a.astype(F32) for a in (x, W1, b1, W2, b2, W3, b3, Wfc, bfc))
    src = edge_index[0].astype(jnp.int32)
    dst = edge_index[1].astype(jnp.int32)
    npad = EP - E
    srcp = jnp.concatenate(
        [src, jnp.zeros((npad,), jnp.int32)]).reshape(NW, NCH, CH)
    dstp = jnp.concatenate(
        [dst, jnp.full((npad,), PAD_BIN, jnp.int32)]).reshape(NW, NCH, CH)
    zhist = jnp.zeros((NBINS,), F32)
    zacc = jnp.zeros((ACC_ROWS, 64), F32)

    odeg_p, ideg_p = _deg_kernel(src, dst, zhist)
    onorm_hb, inorm_hb = _norms(odeg_p, ideg_p)
    onorm = onorm_hb.reshape(NBINS)[:N].reshape(N, 1)
    inorm = inorm_hb.reshape(NBINS)[:N].reshape(N, 1)

    hw = _proj1(x, W1, onorm)
    parts = _agg_kernel(hw, srcp, dstp, zacc)
    hw = _mid(parts, inorm, b1.reshape(1, 64), W2, onorm)
    parts = _agg_kernel(hw, srcp, dstp, zacc)
    hw = _mid(parts, inorm, b2.reshape(1, 64), W3, onorm)
    parts = _agg_kernel(hw, srcp, dstp, zacc)
    out = _head(parts, inorm, b3.reshape(1, 64), Wfc, bfc.reshape(1, 10))
    return out.astype(out_dtype)


# speedup vs baseline: 156.6490x; 1.1604x over previous
import functools

import jax
import jax.numpy as jnp
from jax import lax
from jax.experimental import pallas as pl
from jax.experimental.pallas import tpu as pltpu
from jax.experimental.pallas import tpu_sc as plsc

N = 10000
E = 320000
NW = 32
EW = E // NW
CH = 128
NCH = 80
EWP = CH * NCH
EP = EWP * NW
ACC_ROWS = 10240
PAD_BIN = 10016
HB = 160
RB = 400
F32 = jnp.float32

def _z():
    return jnp.int32(0)


_mesh = plsc.VectorSubcoreMesh(core_axis_name="c", subcore_axis_name="s")


NBINS = HB * 64


@functools.partial(
    pl.kernel,
    mesh=_mesh,
    out_type=[
        jax.ShapeDtypeStruct((NW, NBINS), F32),
        jax.ShapeDtypeStruct((NW, NBINS), F32),
    ],
    scratch_types=[
        pltpu.VMEM((EW,), jnp.int32),
        pltpu.VMEM((EW,), jnp.int32),
        pltpu.VMEM((NBINS,), F32),
        pltpu.VMEM((NBINS,), F32),
    ],
    compiler_params=pltpu.CompilerParams(needs_layout_passes=False),
)
def _deg_kernel(src_hbm, dst_hbm, zeros_hbm, odeg_hbm, ideg_hbm,
                src_v, dst_v, ho_v, hi_v):
    c = lax.axis_index("c")
    s = lax.axis_index("s")
    wid = s * 2 + c
    pltpu.sync_copy(zeros_hbm, ho_v)
    pltpu.sync_copy(zeros_hbm, hi_v)
    pltpu.sync_copy(src_hbm.at[pl.ds(wid * EW, EW)], src_v)
    pltpu.sync_copy(dst_hbm.at[pl.ds(wid * EW, EW)], dst_v)
    ones = jnp.ones((16,), F32)

    def body(i, carry):
        sv = src_v[pl.ds(i * 16, 16)]
        dv = dst_v[pl.ds(i * 16, 16)]
        plsc.addupdate_scatter(ho_v, [sv], ones)
        plsc.addupdate_scatter(hi_v, [dv], ones)
        return carry

    lax.fori_loop(jnp.int32(0), jnp.int32(EW // 16), body, jnp.int32(0))
    pltpu.sync_copy(ho_v, odeg_hbm.at[wid])
    pltpu.sync_copy(hi_v, ideg_hbm.at[wid])


@functools.partial(
    pl.kernel,
    mesh=_mesh,
    out_type=jax.ShapeDtypeStruct((2, ACC_ROWS, 64), F32),
    scratch_types=[
        pltpu.VMEM((NCH, CH), jnp.int32),
        pltpu.VMEM((NCH, CH), jnp.int32),
        pltpu.VMEM((CH, 64), F32),
        pltpu.VMEM((CH, 64), F32),
        pltpu.VMEM_SHARED((ACC_ROWS, 64), F32),
        pltpu.SemaphoreType.DMA,
        pltpu.SemaphoreType.DMA,
    ],
    compiler_params=pltpu.CompilerParams(
        needs_layout_passes=False, use_tc_tiling_on_sc=False),
)
def _agg_kernel(hw_hbm, srcp_hbm, dstp_hbm, zacc_hbm, parts_hbm,
                src_v, dst_v, rows_a, rows_b, acc, sem_a, sem_b):
    c = lax.axis_index("c")
    s = lax.axis_index("s")
    wid = s * 2 + c
    rows_per_tile = ACC_ROWS // 16
    sl = pl.ds(s * rows_per_tile, rows_per_tile)
    pltpu.sync_copy(zacc_hbm.at[sl], acc.at[sl])
    pltpu.sync_copy(srcp_hbm.at[wid], src_v)
    pltpu.sync_copy(dstp_hbm.at[wid], dst_v)
    plsc.subcore_barrier()

    pltpu.async_copy(hw_hbm.at[src_v.at[_z()]], rows_a, sem_a)

    def body(j, carry):
        ja = j * 2
        pltpu.async_copy(hw_hbm.at[src_v.at[ja + 1]], rows_b, sem_b)
        pltpu.make_async_copy(hw_hbm.at[src_v.at[ja]], rows_a, sem_a).wait()
        pltpu.sync_copy(rows_a, acc.at[dst_v.at[ja]], add=True)

        @pl.when(j < NCH // 2 - 1)
        def _():
            pltpu.async_copy(hw_hbm.at[src_v.at[ja + 2]], rows_a, sem_a)

        pltpu.make_async_copy(
            hw_hbm.at[src_v.at[ja + 1]], rows_b, sem_b).wait()
        pltpu.sync_copy(rows_b, acc.at[dst_v.at[ja + 1]], add=True)
        return carry

    lax.fori_loop(jnp.int32(0), jnp.int32(NCH // 2), body, jnp.int32(0))
    plsc.subcore_barrier()
    pltpu.sync_copy(acc.at[sl], parts_hbm.at[c, sl])


def _norms_body(od_ref, id_ref, onorm_ref, inorm_ref):
    od = jnp.sum(od_ref[...], axis=0, keepdims=True)
    idg = jnp.sum(id_ref[...], axis=0, keepdims=True)
    onorm_ref[...] = jnp.where(od > 0, lax.rsqrt(jnp.maximum(od, 1.0)), 0.0)
    inorm_ref[...] = jnp.where(idg > 0, lax.rsqrt(jnp.maximum(idg, 1.0)), 0.0)


def _norms(odeg_p, ideg_p):
    return pl.pallas_call(
        _norms_body,
        out_shape=[
            jax.ShapeDtypeStruct((1, NBINS), F32),
            jax.ShapeDtypeStruct((1, NBINS), F32),
        ],
    )(odeg_p, ideg_p)


def _proj1_body(x_ref, w_ref, on_ref, o_ref):
    o_ref[...] = lax.dot(
        x_ref[...], w_ref[...], preferred_element_type=F32) * on_ref[...]


def _proj1(x, W1, onorm):
    grid = N // RB
    return pl.pallas_call(
        _proj1_body,
        grid=(grid,),
        in_specs=[
            pl.BlockSpec((RB, 128), lambda i: (i, _z())),
            pl.BlockSpec((128, 64), lambda i: (_z(), _z())),
            pl.BlockSpec((RB, 1), lambda i: (i, _z())),
        ],
        out_specs=pl.BlockSpec((RB, 64), lambda i: (i, _z())),
        out_shape=jax.ShapeDtypeStruct((N, 64), F32),
    )(x, W1, onorm)


def _mid_body(p_ref, in_ref, b_ref, w_ref, on_ref, o_ref):
    h = jnp.tanh((p_ref[0] + p_ref[1]) * in_ref[...] + b_ref[...])
    o_ref[...] = lax.dot(
        h, w_ref[...], preferred_element_type=F32) * on_ref[...]


def _mid(parts, inorm, b, W, onorm):
    grid = N // RB
    return pl.pallas_call(
        _mid_body,
        grid=(grid,),
        in_specs=[
            pl.BlockSpec((2, RB, 64), lambda i: (_z(), i, _z())),
            pl.BlockSpec((RB, 1), lambda i: (i, _z())),
            pl.BlockSpec((1, 64), lambda i: (_z(), _z())),
            pl.BlockSpec((64, 64), lambda i: (_z(), _z())),
            pl.BlockSpec((RB, 1), lambda i: (i, _z())),
        ],
        out_specs=pl.BlockSpec((RB, 64), lambda i: (i, _z())),
        out_shape=jax.ShapeDtypeStruct((N, 64), F32),
    )(parts, inorm, b, W, onorm)


def _head_body(p_ref, in_ref, b_ref, w_ref, bf_ref, o_ref):
    h = jnp.tanh((p_ref[0] + p_ref[1]) * in_ref[...] + b_ref[...])
    o_ref[...] = lax.dot(
        h, w_ref[...], preferred_element_type=F32) + bf_ref[...]


def _head(parts, inorm, b, Wfc, bfc):
    grid = N // RB
    return pl.pallas_call(
        _head_body,
        grid=(grid,),
        in_specs=[
            pl.BlockSpec((2, RB, 64), lambda i: (_z(), i, _z())),
            pl.BlockSpec((RB, 1), lambda i: (i, _z())),
            pl.BlockSpec((1, 64), lambda i: (_z(), _z())),
            pl.BlockSpec((64, 10), lambda i: (_z(), _z())),
            pl.BlockSpec((1, 10), lambda i: (_z(), _z())),
        ],
        out_specs=pl.BlockSpec((RB, 10), lambda i: (i, _z())),
        out_shape=jax.ShapeDtypeStruct((N, 10), F32),
    )(parts, inorm, b, Wfc, bfc)


def kernel(x, edge_index, W1, b1, W2, b2, W3, b3, Wfc, bfc):
    out_dtype = jnp.result_type(x.dtype, W1.dtype)
    x, W1, b1, W2, b2, W3, b3, Wfc, bfc = (
        a.astype(F32) for a in (x, W1, b1, W2, b2, W3, b3, Wfc, bfc))
    src = edge_index[0].astype(jnp.int32)
    dst = edge_index[1].astype(jnp.int32)
    npad = EP - E
    srcp = jnp.concatenate(
        [src, jnp.zeros((npad,), jnp.int32)]).reshape(NW, NCH, CH)
    dstp = jnp.concatenate(
        [dst, jnp.full((npad,), PAD_BIN, jnp.int32)]).reshape(NW, NCH, CH)
    zhist = jnp.zeros((NBINS,), F32)
    zacc = jnp.zeros((ACC_ROWS, 64), F32)

    odeg_p, ideg_p = _deg_kernel(src, dst, zhist)
    onorm_hb, inorm_hb = _norms(odeg_p, ideg_p)
    onorm = onorm_hb.reshape(NBINS)[:N].reshape(N, 1)
    inorm = inorm_hb.reshape(NBINS)[:N].reshape(N, 1)

    hw = _proj1(x, W1, onorm)
    parts = _agg_kernel(hw, srcp, dstp, zacc)
    hw = _mid(parts, inorm, b1.reshape(1, 64), W2, onorm)
    parts = _agg_kernel(hw, srcp, dstp, zacc)
    hw = _mid(parts, inorm, b2.reshape(1, 64), W3, onorm)
    parts = _agg_kernel(hw, srcp, dstp, zacc)
    out = _head(parts, inorm, b3.reshape(1, 64), Wfc, bfc.reshape(1, 10))
    return out.astype(out_dtype)


# speedup vs baseline: 308.6470x; 1.9703x over previous
import functools

import jax
import jax.numpy as jnp
from jax import lax
from jax.experimental import pallas as pl
from jax.experimental.pallas import tpu as pltpu
from jax.experimental.pallas import tpu_sc as plsc

N = 10000
E = 320000
NW = 32
EW = E // NW
CH = 128
NCH = 80
EWP = CH * NCH
EP = EWP * NW
ACC_ROWS = 10240
PAD_BIN = 10016
HB = 160
RB = 400
F32 = jnp.float32

def _z():
    return jnp.int32(0)


_mesh = plsc.VectorSubcoreMesh(core_axis_name="c", subcore_axis_name="s")


NBINS = HB * 64


@functools.partial(
    pl.kernel,
    mesh=_mesh,
    out_type=[
        jax.ShapeDtypeStruct((NW, NBINS), F32),
        jax.ShapeDtypeStruct((NW, NBINS), F32),
    ],
    scratch_types=[
        pltpu.VMEM((EW,), jnp.int32),
        pltpu.VMEM((EW,), jnp.int32),
        pltpu.VMEM((NBINS,), F32),
        pltpu.VMEM((NBINS,), F32),
    ],
    compiler_params=pltpu.CompilerParams(needs_layout_passes=False),
)
def _deg_kernel(src_hbm, dst_hbm, zeros_hbm, odeg_hbm, ideg_hbm,
                src_v, dst_v, ho_v, hi_v):
    c = lax.axis_index("c")
    s = lax.axis_index("s")
    wid = s * 2 + c
    pltpu.sync_copy(zeros_hbm, ho_v)
    pltpu.sync_copy(zeros_hbm, hi_v)
    pltpu.sync_copy(src_hbm.at[pl.ds(wid * EW, EW)], src_v)
    pltpu.sync_copy(dst_hbm.at[pl.ds(wid * EW, EW)], dst_v)
    ones = jnp.ones((16,), F32)

    def body(i, carry):
        sv = src_v[pl.ds(i * 16, 16)]
        dv = dst_v[pl.ds(i * 16, 16)]
        plsc.addupdate_scatter(ho_v, [sv], ones)
        plsc.addupdate_scatter(hi_v, [dv], ones)
        return carry

    lax.fori_loop(jnp.int32(0), jnp.int32(EW // 16), body, jnp.int32(0))
    pltpu.sync_copy(ho_v, odeg_hbm.at[wid])
    pltpu.sync_copy(hi_v, ideg_hbm.at[wid])


@functools.partial(
    pl.kernel,
    mesh=_mesh,
    out_type=jax.ShapeDtypeStruct((2, ACC_ROWS, 64), F32),
    scratch_types=[
        pltpu.VMEM((NCH, CH), jnp.int32),
        pltpu.VMEM((NCH, CH), jnp.int32),
        pltpu.VMEM((CH, 64), F32),
        pltpu.VMEM((CH, 64), F32),
        pltpu.VMEM_SHARED((ACC_ROWS, 64), F32),
        pltpu.VMEM_SHARED((N, 64), F32),
        pltpu.SemaphoreType.DMA,
        pltpu.SemaphoreType.DMA,
    ],
    compiler_params=pltpu.CompilerParams(
        needs_layout_passes=False, use_tc_tiling_on_sc=False),
)
def _agg_kernel(hw_hbm, srcp_hbm, dstp_hbm, parts_hbm,
                src_v, dst_v, rows_a, rows_b, acc, hw_s, sem_a, sem_b):
    c = lax.axis_index("c")
    s = lax.axis_index("s")
    wid = s * 2 + c
    rows_per_tile = ACC_ROWS // 16
    sl = pl.ds(s * rows_per_tile, rows_per_tile)
    hw_rows = N // 16
    hsl = pl.ds(s * hw_rows, hw_rows)
    pltpu.async_copy(hw_hbm.at[hsl], hw_s.at[hsl], sem_b)
    pltpu.async_copy(srcp_hbm.at[wid], src_v, sem_a)
    pltpu.async_copy(dstp_hbm.at[wid], dst_v, sem_a)
    z16 = jnp.zeros((16,), F32)

    def zbody(i, carry):
        rows_a[i, pl.ds(0, 16)] = z16
        rows_a[i, pl.ds(16, 16)] = z16
        rows_a[i, pl.ds(32, 16)] = z16
        rows_a[i, pl.ds(48, 16)] = z16
        return carry

    lax.fori_loop(jnp.int32(0), jnp.int32(CH), zbody, jnp.int32(0))
    for k in range(rows_per_tile // CH):
        pltpu.async_copy(
            rows_a, acc.at[pl.ds(s * rows_per_tile + k * CH, CH)], sem_b)
    pltpu.make_async_copy(srcp_hbm.at[wid], src_v, sem_a).wait()
    pltpu.make_async_copy(dstp_hbm.at[wid], dst_v, sem_a).wait()
    pltpu.make_async_copy(hw_hbm.at[hsl], hw_s.at[hsl], sem_b).wait()
    for k in range(rows_per_tile // CH):
        pltpu.make_async_copy(
            rows_a, acc.at[pl.ds(s * rows_per_tile + k * CH, CH)],
            sem_b).wait()
    plsc.subcore_barrier()

    pltpu.async_copy(hw_s.at[src_v.at[_z()]], rows_a, sem_a)

    def body(j, carry):
        ja = j * 2
        pltpu.async_copy(hw_s.at[src_v.at[ja + 1]], rows_b, sem_b)
        pltpu.make_async_copy(hw_s.at[src_v.at[ja]], rows_a, sem_a).wait()
        pltpu.sync_copy(rows_a, acc.at[dst_v.at[ja]], add=True)

        @pl.when(j < NCH // 2 - 1)
        def _():
            pltpu.async_copy(hw_s.at[src_v.at[ja + 2]], rows_a, sem_a)

        pltpu.make_async_copy(
            hw_s.at[src_v.at[ja + 1]], rows_b, sem_b).wait()
        pltpu.sync_copy(rows_b, acc.at[dst_v.at[ja + 1]], add=True)
        return carry

    lax.fori_loop(jnp.int32(0), jnp.int32(NCH // 2), body, jnp.int32(0))
    plsc.subcore_barrier()
    pltpu.sync_copy(acc.at[sl], parts_hbm.at[c, sl])


def _norms_body(od_ref, id_ref, onorm_ref, inorm_ref):
    od = jnp.sum(od_ref[...], axis=0, keepdims=True)
    idg = jnp.sum(id_ref[...], axis=0, keepdims=True)
    onorm_ref[...] = jnp.where(od > 0, lax.rsqrt(jnp.maximum(od, 1.0)), 0.0)
    inorm_ref[...] = jnp.where(idg > 0, lax.rsqrt(jnp.maximum(idg, 1.0)), 0.0)


def _norms(odeg_p, ideg_p):
    return pl.pallas_call(
        _norms_body,
        out_shape=[
            jax.ShapeDtypeStruct((1, NBINS), F32),
            jax.ShapeDtypeStruct((1, NBINS), F32),
        ],
    )(odeg_p, ideg_p)


def _proj1_body(x_ref, w_ref, on_ref, o_ref):
    o_ref[...] = lax.dot(
        x_ref[...], w_ref[...], preferred_element_type=F32) * on_ref[...]


def _proj1(x, W1, onorm):
    grid = N // RB
    return pl.pallas_call(
        _proj1_body,
        grid=(grid,),
        in_specs=[
            pl.BlockSpec((RB, 128), lambda i: (i, _z())),
            pl.BlockSpec((128, 64), lambda i: (_z(), _z())),
            pl.BlockSpec((RB, 1), lambda i: (i, _z())),
        ],
        out_specs=pl.BlockSpec((RB, 64), lambda i: (i, _z())),
        out_shape=jax.ShapeDtypeStruct((N, 64), F32),
    )(x, W1, onorm)


def _mid_body(p_ref, in_ref, b_ref, w_ref, on_ref, o_ref):
    h = jnp.tanh((p_ref[0] + p_ref[1]) * in_ref[...] + b_ref[...])
    o_ref[...] = lax.dot(
        h, w_ref[...], preferred_element_type=F32) * on_ref[...]


def _mid(parts, inorm, b, W, onorm):
    grid = N // RB
    return pl.pallas_call(
        _mid_body,
        grid=(grid,),
        in_specs=[
            pl.BlockSpec((2, RB, 64), lambda i: (_z(), i, _z())),
            pl.BlockSpec((RB, 1), lambda i: (i, _z())),
            pl.BlockSpec((1, 64), lambda i: (_z(), _z())),
            pl.BlockSpec((64, 64), lambda i: (_z(), _z())),
            pl.BlockSpec((RB, 1), lambda i: (i, _z())),
        ],
        out_specs=pl.BlockSpec((RB, 64), lambda i: (i, _z())),
        out_shape=jax.ShapeDtypeStruct((N, 64), F32),
    )(parts, inorm, b, W, onorm)


def _head_body(p_ref, in_ref, b_ref, w_ref, bf_ref, o_ref):
    h = jnp.tanh((p_ref[0] + p_ref[1]) * in_ref[...] + b_ref[...])
    o_ref[...] = lax.dot(
        h, w_ref[...], preferred_element_type=F32) + bf_ref[...]


def _head(parts, inorm, b, Wfc, bfc):
    grid = N // RB
    return pl.pallas_call(
        _head_body,
        grid=(grid,),
        in_specs=[
            pl.BlockSpec((2, RB, 64), lambda i: (_z(), i, _z())),
            pl.BlockSpec((RB, 1), lambda i: (i, _z())),
            pl.BlockSpec((1, 64), lambda i: (_z(), _z())),
            pl.BlockSpec((64, 10), lambda i: (_z(), _z())),
            pl.BlockSpec((1, 10), lambda i: (_z(), _z())),
        ],
        out_specs=pl.BlockSpec((RB, 10), lambda i: (i, _z())),
        out_shape=jax.ShapeDtypeStruct((N, 10), F32),
    )(parts, inorm, b, Wfc, bfc)


def kernel(x, edge_index, W1, b1, W2, b2, W3, b3, Wfc, bfc):
    out_dtype = jnp.result_type(x.dtype, W1.dtype)
    x, W1, b1, W2, b2, W3, b3, Wfc, bfc = (
        a.astype(F32) for a in (x, W1, b1, W2, b2, W3, b3, Wfc, bfc))
    src = edge_index[0].astype(jnp.int32)
    dst = edge_index[1].astype(jnp.int32)
    npad = EP - E
    srcp = jnp.concatenate(
        [src, jnp.zeros((npad,), jnp.int32)]).reshape(NW, NCH, CH)
    dstp = jnp.concatenate(
        [dst, jnp.full((npad,), PAD_BIN, jnp.int32)]).reshape(NW, NCH, CH)
    zhist = jnp.zeros((NBINS,), F32)

    odeg_p, ideg_p = _deg_kernel(src, dst, zhist)
    onorm_hb, inorm_hb = _norms(odeg_p, ideg_p)
    onorm = onorm_hb.reshape(NBINS)[:N].reshape(N, 1)
    inorm = inorm_hb.reshape(NBINS)[:N].reshape(N, 1)

    hw = _proj1(x, W1, onorm)
    parts = _agg_kernel(hw, srcp, dstp)
    hw = _mid(parts, inorm, b1.reshape(1, 64), W2, onorm)
    parts = _agg_kernel(hw, srcp, dstp)
    hw = _mid(parts, inorm, b2.reshape(1, 64), W3, onorm)
    parts = _agg_kernel(hw, srcp, dstp)
    out = _head(parts, inorm, b3.reshape(1, 64), Wfc, bfc.reshape(1, 10))
    return out.astype(out_dtype)


# speedup vs baseline: 344.0128x; 1.1146x over previous
import functools

import jax
import jax.numpy as jnp
from jax import lax
from jax.experimental import pallas as pl
from jax.experimental.pallas import tpu as pltpu
from jax.experimental.pallas import tpu_sc as plsc

N = 10000
E = 320000
NW = 32
EW = E // NW
CH = 128
NFULL = EW // CH
TAIL = EW - NFULL * CH
ACC_ROWS = 10240
HB = 160
RB = 2000
F32 = jnp.float32

def _z():
    return jnp.int32(0)


_mesh = plsc.VectorSubcoreMesh(core_axis_name="c", subcore_axis_name="s")


NBINS = HB * 64


@functools.partial(
    pl.kernel,
    mesh=_mesh,
    out_type=[
        jax.ShapeDtypeStruct((NW, NBINS), F32),
        jax.ShapeDtypeStruct((NW, NBINS), F32),
    ],
    scratch_types=[
        pltpu.VMEM((EW,), jnp.int32),
        pltpu.VMEM((EW,), jnp.int32),
        pltpu.VMEM((NBINS,), F32),
        pltpu.VMEM((NBINS,), F32),
    ],
    compiler_params=pltpu.CompilerParams(needs_layout_passes=False),
)
def _deg_kernel(src_hbm, dst_hbm, zeros_hbm, odeg_hbm, ideg_hbm,
                src_v, dst_v, ho_v, hi_v):
    c = lax.axis_index("c")
    s = lax.axis_index("s")
    wid = s * 2 + c
    pltpu.sync_copy(zeros_hbm, ho_v)
    pltpu.sync_copy(zeros_hbm, hi_v)
    pltpu.sync_copy(src_hbm.at[pl.ds(wid * EW, EW)], src_v)
    pltpu.sync_copy(dst_hbm.at[pl.ds(wid * EW, EW)], dst_v)
    ones = jnp.ones((16,), F32)

    def body(i, carry):
        sv = src_v[pl.ds(i * 16, 16)]
        dv = dst_v[pl.ds(i * 16, 16)]
        plsc.addupdate_scatter(ho_v, [sv], ones)
        plsc.addupdate_scatter(hi_v, [dv], ones)
        return carry

    lax.fori_loop(jnp.int32(0), jnp.int32(EW // 16), body, jnp.int32(0))
    pltpu.sync_copy(ho_v, odeg_hbm.at[wid])
    pltpu.sync_copy(hi_v, ideg_hbm.at[wid])


@functools.partial(
    pl.kernel,
    mesh=_mesh,
    out_type=jax.ShapeDtypeStruct((2, ACC_ROWS, 64), F32),
    scratch_types=[
        pltpu.VMEM((EW,), jnp.int32),
        pltpu.VMEM((EW,), jnp.int32),
        pltpu.VMEM((CH, 64), F32),
        pltpu.VMEM((CH, 64), F32),
        pltpu.VMEM_SHARED((ACC_ROWS, 64), F32),
        pltpu.VMEM_SHARED((N, 64), F32),
        pltpu.SemaphoreType.DMA,
        pltpu.SemaphoreType.DMA,
    ],
    compiler_params=pltpu.CompilerParams(
        needs_layout_passes=False, use_tc_tiling_on_sc=False),
)
def _agg_kernel(hw_hbm, srcp_hbm, dstp_hbm, parts_hbm,
                src_v, dst_v, rows_a, rows_b, acc, hw_s, sem_a, sem_b):
    c = lax.axis_index("c")
    s = lax.axis_index("s")
    wid = s * 2 + c
    rows_per_tile = ACC_ROWS // 16
    sl = pl.ds(s * rows_per_tile, rows_per_tile)
    hw_rows = N // 16
    hsl = pl.ds(s * hw_rows, hw_rows)
    pltpu.async_copy(hw_hbm.at[hsl], hw_s.at[hsl], sem_b)
    pltpu.async_copy(srcp_hbm.at[wid], src_v, sem_a)
    pltpu.async_copy(dstp_hbm.at[wid], dst_v, sem_a)
    z16 = jnp.zeros((16,), F32)

    def zbody(i, carry):
        rows_a[i, pl.ds(0, 16)] = z16
        rows_a[i, pl.ds(16, 16)] = z16
        rows_a[i, pl.ds(32, 16)] = z16
        rows_a[i, pl.ds(48, 16)] = z16
        return carry

    lax.fori_loop(jnp.int32(0), jnp.int32(CH), zbody, jnp.int32(0))
    for k in range(rows_per_tile // CH):
        pltpu.async_copy(
            rows_a, acc.at[pl.ds(s * rows_per_tile + k * CH, CH)], sem_b)
    pltpu.make_async_copy(srcp_hbm.at[wid], src_v, sem_a).wait()
    pltpu.make_async_copy(dstp_hbm.at[wid], dst_v, sem_a).wait()
    pltpu.make_async_copy(hw_hbm.at[hsl], hw_s.at[hsl], sem_b).wait()
    for k in range(rows_per_tile // CH):
        pltpu.make_async_copy(
            rows_a, acc.at[pl.ds(s * rows_per_tile + k * CH, CH)],
            sem_b).wait()
    plsc.subcore_barrier()

    def _idx(j):
        return src_v.at[pl.ds(j * CH, CH)]

    def _dsti(j):
        return dst_v.at[pl.ds(j * CH, CH)]

    pltpu.async_copy(hw_s.at[_idx(_z())], rows_a, sem_a)

    def body(j, carry):
        ja = j * 2
        pltpu.async_copy(hw_s.at[_idx(ja + 1)], rows_b, sem_b)
        pltpu.make_async_copy(hw_s.at[_idx(ja)], rows_a, sem_a).wait()
        pltpu.sync_copy(rows_a, acc.at[_dsti(ja)], add=True)

        @pl.when(j < NFULL // 2 - 1)
        def _():
            pltpu.async_copy(hw_s.at[_idx(ja + 2)], rows_a, sem_a)

        pltpu.make_async_copy(hw_s.at[_idx(ja + 1)], rows_b, sem_b).wait()
        pltpu.sync_copy(rows_b, acc.at[_dsti(ja + 1)], add=True)
        return carry

    lax.fori_loop(jnp.int32(0), jnp.int32(NFULL // 2), body, jnp.int32(0))
    tsl = pl.ds(NFULL * CH, TAIL)
    rsl = pl.ds(0, TAIL)
    pltpu.sync_copy(hw_s.at[src_v.at[tsl]], rows_a.at[rsl])
    pltpu.sync_copy(rows_a.at[rsl], acc.at[dst_v.at[tsl]], add=True)
    plsc.subcore_barrier()
    pltpu.sync_copy(acc.at[sl], parts_hbm.at[c, sl])


def _norms_body(od_ref, id_ref, onorm_ref, inorm_ref):
    od = jnp.sum(od_ref[...], axis=0, keepdims=True)
    idg = jnp.sum(id_ref[...], axis=0, keepdims=True)
    onorm_ref[...] = jnp.where(od > 0, lax.rsqrt(jnp.maximum(od, 1.0)), 0.0)
    inorm_ref[...] = jnp.where(idg > 0, lax.rsqrt(jnp.maximum(idg, 1.0)), 0.0)


def _norms(odeg_p, ideg_p):
    return pl.pallas_call(
        _norms_body,
        out_shape=[
            jax.ShapeDtypeStruct((1, NBINS), F32),
            jax.ShapeDtypeStruct((1, NBINS), F32),
        ],
    )(odeg_p, ideg_p)


def _proj1_body(x_ref, w_ref, on_ref, o_ref):
    o_ref[...] = lax.dot(
        x_ref[...], w_ref[...], preferred_element_type=F32) * on_ref[...]


def _proj1(x, W1, onorm):
    grid = N // RB
    return pl.pallas_call(
        _proj1_body,
        grid=(grid,),
        in_specs=[
            pl.BlockSpec((RB, 128), lambda i: (i, _z())),
            pl.BlockSpec((128, 64), lambda i: (_z(), _z())),
            pl.BlockSpec((RB, 1), lambda i: (i, _z())),
        ],
        out_specs=pl.BlockSpec((RB, 64), lambda i: (i, _z())),
        out_shape=jax.ShapeDtypeStruct((N, 64), F32),
    )(x, W1, onorm)


def _mid_body(p_ref, in_ref, b_ref, w_ref, on_ref, o_ref):
    h = jnp.tanh((p_ref[0] + p_ref[1]) * in_ref[...] + b_ref[...])
    o_ref[...] = lax.dot(
        h, w_ref[...], preferred_element_type=F32) * on_ref[...]


def _mid(parts, inorm, b, W, onorm):
    grid = N // RB
    return pl.pallas_call(
        _mid_body,
        grid=(grid,),
        in_specs=[
            pl.BlockSpec((2, RB, 64), lambda i: (_z(), i, _z())),
            pl.BlockSpec((RB, 1), lambda i: (i, _z())),
            pl.BlockSpec((1, 64), lambda i: (_z(), _z())),
            pl.BlockSpec((64, 64), lambda i: (_z(), _z())),
            pl.BlockSpec((RB, 1), lambda i: (i, _z())),
        ],
        out_specs=pl.BlockSpec((RB, 64), lambda i: (i, _z())),
        out_shape=jax.ShapeDtypeStruct((N, 64), F32),
    )(parts, inorm, b, W, onorm)


def _head_body(p_ref, in_ref, b_ref, w_ref, bf_ref, o_ref):
    h = jnp.tanh((p_ref[0] + p_ref[1]) * in_ref[...] + b_ref[...])
    o_ref[...] = lax.dot(
        h, w_ref[...], preferred_element_type=F32) + bf_ref[...]


def _head(parts, inorm, b, Wfc, bfc):
    grid = N // RB
    return pl.pallas_call(
        _head_body,
        grid=(grid,),
        in_specs=[
            pl.BlockSpec((2, RB, 64), lambda i: (_z(), i, _z())),
            pl.BlockSpec((RB, 1), lambda i: (i, _z())),
            pl.BlockSpec((1, 64), lambda i: (_z(), _z())),
            pl.BlockSpec((64, 10), lambda i: (_z(), _z())),
            pl.BlockSpec((1, 10), lambda i: (_z(), _z())),
        ],
        out_specs=pl.BlockSpec((RB, 10), lambda i: (i, _z())),
        out_shape=jax.ShapeDtypeStruct((N, 10), F32),
    )(parts, inorm, b, Wfc, bfc)


def kernel(x, edge_index, W1, b1, W2, b2, W3, b3, Wfc, bfc):
    out_dtype = jnp.result_type(x.dtype, W1.dtype)
    x, W1, b1, W2, b2, W3, b3, Wfc, bfc = (
        a.astype(F32) for a in (x, W1, b1, W2, b2, W3, b3, Wfc, bfc))
    src = edge_index[0].astype(jnp.int32)
    dst = edge_index[1].astype(jnp.int32)
    srcp = src.reshape(NW, EW)
    dstp = dst.reshape(NW, EW)
    zhist = jnp.zeros((NBINS,), F32)

    odeg_p, ideg_p = _deg_kernel(src, dst, zhist)
    onorm_hb, inorm_hb = _norms(odeg_p, ideg_p)
    onorm = onorm_hb.reshape(NBINS)[:N].reshape(N, 1)
    inorm = inorm_hb.reshape(NBINS)[:N].reshape(N, 1)

    hw = _proj1(x, W1, onorm)
    parts = _agg_kernel(hw, srcp, dstp)
    hw = _mid(parts, inorm, b1.reshape(1, 64), W2, onorm)
    parts = _agg_kernel(hw, srcp, dstp)
    hw = _mid(parts, inorm, b2.reshape(1, 64), W3, onorm)
    parts = _agg_kernel(hw, srcp, dstp)
    out = _head(parts, inorm, b3.reshape(1, 64), Wfc, bfc.reshape(1, 10))
    return out.astype(out_dtype)


# speedup vs baseline: 371.5961x; 1.0802x over previous
import functools

import jax
import jax.numpy as jnp
from jax import lax
from jax.experimental import pallas as pl
from jax.experimental.pallas import tpu as pltpu
from jax.experimental.pallas import tpu_sc as plsc

N = 10000
E = 320000
NW = 32
EW = E // NW
CH = 128
NFULL = EW // CH
TAIL = EW - NFULL * CH
ACC_ROWS = 10240
HB = 160
RB = 2000
F32 = jnp.float32

def _z():
    return jnp.int32(0)


_mesh = plsc.VectorSubcoreMesh(core_axis_name="c", subcore_axis_name="s")


NBINS = HB * 64


@functools.partial(
    pl.kernel,
    mesh=_mesh,
    out_type=[
        jax.ShapeDtypeStruct((NW, NBINS), F32),
        jax.ShapeDtypeStruct((NW, NBINS), F32),
    ],
    scratch_types=[
        pltpu.VMEM((EW,), jnp.int32),
        pltpu.VMEM((EW,), jnp.int32),
        pltpu.VMEM((NBINS,), F32),
        pltpu.VMEM((NBINS,), F32),
    ],
    compiler_params=pltpu.CompilerParams(needs_layout_passes=False),
)
def _deg_kernel(src_hbm, dst_hbm, zeros_hbm, odeg_hbm, ideg_hbm,
                src_v, dst_v, ho_v, hi_v):
    c = lax.axis_index("c")
    s = lax.axis_index("s")
    wid = s * 2 + c
    pltpu.sync_copy(zeros_hbm, ho_v)
    pltpu.sync_copy(zeros_hbm, hi_v)
    pltpu.sync_copy(src_hbm.at[pl.ds(wid * EW, EW)], src_v)
    pltpu.sync_copy(dst_hbm.at[pl.ds(wid * EW, EW)], dst_v)
    ones = jnp.ones((16,), F32)

    def body(i, carry):
        sv = src_v[pl.ds(i * 16, 16)]
        dv = dst_v[pl.ds(i * 16, 16)]
        plsc.addupdate_scatter(ho_v, [sv], ones)
        plsc.addupdate_scatter(hi_v, [dv], ones)
        return carry

    lax.fori_loop(jnp.int32(0), jnp.int32(EW // 16), body, jnp.int32(0))
    pltpu.sync_copy(ho_v, odeg_hbm.at[wid])
    pltpu.sync_copy(hi_v, ideg_hbm.at[wid])


@functools.partial(
    pl.kernel,
    mesh=_mesh,
    out_type=jax.ShapeDtypeStruct((2, ACC_ROWS, 64), F32),
    scratch_types=[
        pltpu.VMEM((EW,), jnp.int32),
        pltpu.VMEM((EW,), jnp.int32),
        pltpu.VMEM((CH, 64), F32),
        pltpu.VMEM((CH, 64), F32),
        pltpu.VMEM_SHARED((ACC_ROWS, 64), F32),
        pltpu.VMEM_SHARED((N, 64), F32),
        pltpu.SemaphoreType.DMA,
        pltpu.SemaphoreType.DMA,
    ],
    compiler_params=pltpu.CompilerParams(
        needs_layout_passes=False, use_tc_tiling_on_sc=False),
)
def _agg_kernel(hw_hbm, srcp_hbm, dstp_hbm, parts_hbm,
                src_v, dst_v, rows_a, rows_b, acc, hw_s, sem_a, sem_b):
    c = lax.axis_index("c")
    s = lax.axis_index("s")
    wid = s * 2 + c
    rows_per_tile = ACC_ROWS // 16
    sl = pl.ds(s * rows_per_tile, rows_per_tile)
    hw_rows = N // 16
    hsl = pl.ds(s * hw_rows, hw_rows)
    pltpu.async_copy(hw_hbm.at[hsl], hw_s.at[hsl], sem_b)
    pltpu.async_copy(srcp_hbm.at[wid], src_v, sem_a)
    pltpu.async_copy(dstp_hbm.at[wid], dst_v, sem_a)
    z16 = jnp.zeros((16,), F32)

    def zbody(i, carry):
        rows_a[i, pl.ds(0, 16)] = z16
        rows_a[i, pl.ds(16, 16)] = z16
        rows_a[i, pl.ds(32, 16)] = z16
        rows_a[i, pl.ds(48, 16)] = z16
        return carry

    lax.fori_loop(jnp.int32(0), jnp.int32(CH), zbody, jnp.int32(0))
    for k in range(rows_per_tile // CH):
        pltpu.async_copy(
            rows_a, acc.at[pl.ds(s * rows_per_tile + k * CH, CH)], sem_b)
    pltpu.make_async_copy(srcp_hbm.at[wid], src_v, sem_a).wait()
    pltpu.make_async_copy(dstp_hbm.at[wid], dst_v, sem_a).wait()
    pltpu.make_async_copy(hw_hbm.at[hsl], hw_s.at[hsl], sem_b).wait()
    for k in range(rows_per_tile // CH):
        pltpu.make_async_copy(
            rows_a, acc.at[pl.ds(s * rows_per_tile + k * CH, CH)],
            sem_b).wait()
    plsc.subcore_barrier()

    def _idx(j):
        return src_v.at[pl.ds(j * CH, CH)]

    def _dsti(j):
        return dst_v.at[pl.ds(j * CH, CH)]

    pltpu.async_copy(hw_s.at[_idx(_z())], rows_a, sem_a)

    def body(j, carry):
        ja = j * 2
        pltpu.async_copy(hw_s.at[_idx(ja + 1)], rows_b, sem_b)
        pltpu.make_async_copy(hw_s.at[_idx(ja)], rows_a, sem_a).wait()
        pltpu.sync_copy(rows_a, acc.at[_dsti(ja)], add=True)

        @pl.when(j < NFULL // 2 - 1)
        def _():
            pltpu.async_copy(hw_s.at[_idx(ja + 2)], rows_a, sem_a)

        pltpu.make_async_copy(hw_s.at[_idx(ja + 1)], rows_b, sem_b).wait()
        pltpu.sync_copy(rows_b, acc.at[_dsti(ja + 1)], add=True)
        return carry

    lax.fori_loop(jnp.int32(0), jnp.int32(NFULL // 2), body, jnp.int32(0))
    tsl = pl.ds(NFULL * CH, TAIL)
    rsl = pl.ds(0, TAIL)
    pltpu.sync_copy(hw_s.at[src_v.at[tsl]], rows_a.at[rsl])
    pltpu.sync_copy(rows_a.at[rsl], acc.at[dst_v.at[tsl]], add=True)
    plsc.subcore_barrier()
    pltpu.sync_copy(acc.at[sl], parts_hbm.at[c, sl])


def _norms_body(od_ref, id_ref, onorm_ref, inorm_ref):
    od = jnp.sum(od_ref[...], axis=0, keepdims=True)
    idg = jnp.sum(id_ref[...], axis=0, keepdims=True)
    onorm_ref[...] = jnp.where(od > 0, lax.rsqrt(jnp.maximum(od, 1.0)), 0.0)
    inorm_ref[...] = jnp.where(idg > 0, lax.rsqrt(jnp.maximum(idg, 1.0)), 0.0)


def _norms(odeg_p, ideg_p):
    return pl.pallas_call(
        _norms_body,
        out_shape=[
            jax.ShapeDtypeStruct((1, NBINS), F32),
            jax.ShapeDtypeStruct((1, NBINS), F32),
        ],
    )(odeg_p, ideg_p)


def _proj1_body(x_ref, w_ref, on_ref, o_ref):
    o_ref[...] = lax.dot(
        x_ref[...], w_ref[...], preferred_element_type=F32) * on_ref[...]


def _proj1(x, W1, onorm):
    grid = N // RB
    return pl.pallas_call(
        _proj1_body,
        grid=(grid,),
        in_specs=[
            pl.BlockSpec((RB, 128), lambda i: (i, _z())),
            pl.BlockSpec((128, 64), lambda i: (_z(), _z())),
            pl.BlockSpec((RB, 1), lambda i: (i, _z())),
        ],
        out_specs=pl.BlockSpec((RB, 64), lambda i: (i, _z())),
        out_shape=jax.ShapeDtypeStruct((N, 64), F32),
    )(x, W1, onorm)


def _mid_body(p_ref, in_ref, b_ref, w_ref, on_ref, o_ref):
    h = jnp.tanh((p_ref[0] + p_ref[1]) * in_ref[...] + b_ref[...])
    o_ref[...] = lax.dot(
        h, w_ref[...], preferred_element_type=F32) * on_ref[...]


def _mid(parts, inorm, b, W, onorm):
    grid = N // RB
    return pl.pallas_call(
        _mid_body,
        grid=(grid,),
        in_specs=[
            pl.BlockSpec((2, RB, 64), lambda i: (_z(), i, _z())),
            pl.BlockSpec((RB, 1), lambda i: (i, _z())),
            pl.BlockSpec((1, 64), lambda i: (_z(), _z())),
            pl.BlockSpec((64, 64), lambda i: (_z(), _z())),
            pl.BlockSpec((RB, 1), lambda i: (i, _z())),
        ],
        out_specs=pl.BlockSpec((RB, 64), lambda i: (i, _z())),
        out_shape=jax.ShapeDtypeStruct((N, 64), F32),
    )(parts, inorm, b, W, onorm)


def _head_body(p_ref, in_ref, b_ref, w_ref, bf_ref, hi_ref, lo_ref):
    h = jnp.tanh((p_ref[0] + p_ref[1]) * in_ref[...] + b_ref[...])
    y = lax.dot(h, w_ref[...], preferred_element_type=F32) + bf_ref[...]
    bits = lax.bitcast_convert_type(y, jnp.uint32)
    sign = bits & jnp.uint32(0x80000000)
    e = (bits >> jnp.uint32(23)) & jnp.uint32(0xFF)
    m = bits & jnp.uint32(0x7FFFFF)
    hi = sign | ((e + jnp.uint32(896)) << jnp.uint32(20)) | (
        m >> jnp.uint32(3))
    nz = e != jnp.uint32(0)
    hi_ref[...] = jnp.where(nz, hi, sign)
    lo_ref[...] = jnp.where(nz, m << jnp.uint32(29), jnp.uint32(0))


def _head(parts, inorm, b, Wfc, bfc):
    grid = N // RB
    return pl.pallas_call(
        _head_body,
        grid=(grid,),
        in_specs=[
            pl.BlockSpec((2, RB, 64), lambda i: (_z(), i, _z())),
            pl.BlockSpec((RB, 1), lambda i: (i, _z())),
            pl.BlockSpec((1, 64), lambda i: (_z(), _z())),
            pl.BlockSpec((64, 10), lambda i: (_z(), _z())),
            pl.BlockSpec((1, 10), lambda i: (_z(), _z())),
        ],
        out_specs=[
            pl.BlockSpec((RB, 10), lambda i: (i, _z())),
            pl.BlockSpec((RB, 10), lambda i: (i, _z())),
        ],
        out_shape=[
            jax.ShapeDtypeStruct((N, 10), jnp.uint32),
            jax.ShapeDtypeStruct((N, 10), jnp.uint32),
        ],
    )(parts, inorm, b, Wfc, bfc)


def kernel(x, edge_index, W1, b1, W2, b2, W3, b3, Wfc, bfc):
    out_dtype = jnp.result_type(x.dtype, W1.dtype)
    x, W1, b1, W2, b2, W3, b3, Wfc, bfc = (
        a.astype(F32) for a in (x, W1, b1, W2, b2, W3, b3, Wfc, bfc))
    src = edge_index[0].astype(jnp.int32)
    dst = edge_index[1].astype(jnp.int32)
    srcp = src.reshape(NW, EW)
    dstp = dst.reshape(NW, EW)
    zhist = jnp.zeros((NBINS,), F32)

    odeg_p, ideg_p = _deg_kernel(src, dst, zhist)
    onorm_hb, inorm_hb = _norms(odeg_p, ideg_p)
    onorm = onorm_hb.reshape(NBINS)[:N].reshape(N, 1)
    inorm = inorm_hb.reshape(NBINS)[:N].reshape(N, 1)

    hw = _proj1(x, W1, onorm)
    parts = _agg_kernel(hw, srcp, dstp)
    hw = _mid(parts, inorm, b1.reshape(1, 64), W2, onorm)
    parts = _agg_kernel(hw, srcp, dstp)
    hw = _mid(parts, inorm, b2.reshape(1, 64), W3, onorm)
    parts = _agg_kernel(hw, srcp, dstp)
    hi, lo = _head(parts, inorm, b3.reshape(1, 64), Wfc, bfc.reshape(1, 10))
    return lax.bitcast_convert_type(
        jnp.stack([lo, hi], axis=-1), jnp.float64).astype(out_dtype)
